# baseline (speedup 1.0000x reference)
import functools

import jax
import jax.numpy as jnp
from jax import lax
from jax.experimental import pallas as pl
from jax.experimental.pallas import tpu as pltpu
from jax.experimental.pallas import tpu_sc as plsc

N = 10000
D = 128
E = 320000

NC = 2
NS = 16
DH = D // NC
C = 128

EPW = 20480
E_PAD = EPW * NS
NCHUNK = EPW // C
NBUF = 4
ROWS = 10240
ROWS_PER_SUB = ROWS // NS
ZROWS = 128

_mesh = plsc.VectorSubcoreMesh(
    core_axis_name="c", subcore_axis_name="s", num_cores=NC, num_subcores=NS
)


@functools.partial(
    pl.kernel,
    out_type=jax.ShapeDtypeStruct((NC, ROWS, DH), jnp.float32),
    mesh=_mesh,
    scratch_types=[
        pltpu.VMEM((NCHUNK, C), jnp.int32),
        pltpu.VMEM((NCHUNK, C), jnp.int32),
        pltpu.VMEM((NBUF, C, DH), jnp.bfloat16),
        pltpu.VMEM((C, DH), jnp.float32),
        pltpu.VMEM((ZROWS, DH), jnp.float32),
        pltpu.VMEM_SHARED((ROWS, DH), jnp.float32),
        pltpu.SemaphoreType.DMA,
        pltpu.SemaphoreType.DMA,
        pltpu.SemaphoreType.DMA,
        pltpu.SemaphoreType.DMA,
        pltpu.SemaphoreType.DMA,
        pltpu.SemaphoreType.DMA,
    ],
    compiler_params=pltpu.CompilerParams(
        use_tc_tiling_on_sc=False, needs_layout_passes=False),
)
def _sc_aggregate(h_hbm, src_hbm, dst_hbm, out_hbm,
                  idxs_v, idxd_v, rows_bf, rows_f, zero_v, acc_sh,
                  sem0, sem1, sem2, sem3, isem_s, isem_d):
    sems = (sem0, sem1, sem2, sem3)
    ci = lax.axis_index("c")
    si = lax.axis_index("s")
    h_half = h_hbm.at[ci]
    sb0 = si * NCHUNK

    idx_s_cp = pltpu.make_async_copy(
        src_hbm.at[pl.ds(sb0, NCHUNK)], idxs_v, isem_s)
    idx_d_cp = pltpu.make_async_copy(
        dst_hbm.at[pl.ds(sb0, NCHUNK)], idxd_v, isem_d)
    idx_s_cp.start()
    idx_d_cp.start()

    zeros16 = jnp.zeros((16,), jnp.float32)

    @pl.loop(0, ZROWS)
    def _(i):
        @pl.loop(0, DH // 16)
        def _(j):
            zero_v[i, pl.ds(j * 16, 16)] = zeros16

    row0 = si * ROWS_PER_SUB

    @pl.loop(0, ROWS_PER_SUB // ZROWS)
    def _(k):
        pltpu.sync_copy(zero_v, acc_sh.at[pl.ds(row0 + k * ZROWS, ZROWS)])

    plsc.subcore_barrier()
    idx_s_cp.wait()
    idx_d_cp.wait()

    def start_gather(b, t):
        pltpu.async_copy(h_half.at[idxs_v.at[t]], rows_bf.at[b], sems[b])

    def finish_chunk(b, t):
        pltpu.make_async_copy(
            h_half.at[idxs_v.at[t]], rows_bf.at[b], sems[b]).wait()

        @pl.loop(0, C)
        def _(r):
            for g in range(DH // 32):
                lo, hi = plsc.unpack(
                    rows_bf[b, r, pl.ds(g * 32, 32)],
                    format=plsc.PackFormat.INTERLEAVED,
                    preferred_element_type=jnp.float32)
                rows_f[r, pl.ds(g * 32, 16)] = lo
                rows_f[r, pl.ds(g * 32 + 16, 16)] = hi

        pltpu.sync_copy(rows_f, acc_sh.at[idxd_v.at[t]], add=True)

    for b in range(NBUF - 1):
        start_gather(b, b)

    @pl.loop(0, NCHUNK // NBUF)
    def _(g):
        t = g * NBUF
        for k in range(NBUF):
            f = t + k + NBUF - 1

            @pl.when(f < NCHUNK)
            def _():
                start_gather((k + NBUF - 1) % NBUF, f)

            finish_chunk(k, t + k)

    plsc.subcore_barrier()
    pltpu.sync_copy(acc_sh.at[pl.ds(row0, ROWS_PER_SUB)],
                    out_hbm.at[ci].at[pl.ds(row0, ROWS_PER_SUB)])


def _interleave(y):
    b, k = y.shape
    return y.reshape(b, k // 32, 2, 16).swapaxes(2, 3).reshape(b, k)


_BLK = 1024


def _dense_body(act, p_ref, w_ref, b_ref, o_ref):
    y = jnp.concatenate([p_ref[0], p_ref[1]], axis=1)
    y = lax.dot_general(y, w_ref[...], (((1,), (0,)), ((), ())),
                        precision=lax.Precision.HIGHEST,
                        preferred_element_type=jnp.float32)
    y = y + b_ref[...]
    if act == "relu":
        y = _interleave(jnp.maximum(y, 0.0).astype(jnp.bfloat16))
        o_ref[0], o_ref[1] = y[:, :DH], y[:, DH:]
    else:
        m = jnp.max(y, axis=-1, keepdims=True)
        s = jnp.sum(jnp.exp(y - m), axis=-1, keepdims=True)
        o_ref[...] = y - (m + jnp.log(s))


def _dense_relu(p, w, b):
    return pl.pallas_call(
        functools.partial(_dense_body, "relu"),
        grid=(ROWS // _BLK,),
        in_specs=[
            pl.BlockSpec((NC, _BLK, DH), lambda i: (0, i, 0)),
            pl.BlockSpec((D, D), lambda i: (0, 0)),
            pl.BlockSpec((1, D), lambda i: (0, 0)),
        ],
        out_specs=pl.BlockSpec((NC, _BLK, DH), lambda i: (0, i, 0)),
        out_shape=jax.ShapeDtypeStruct((NC, ROWS, DH), jnp.bfloat16),
    )(p, w, b.reshape(1, D))


_FBLK = 1000


def _dense_logsoftmax(p, w, b):
    return pl.pallas_call(
        functools.partial(_dense_body, "logsoftmax"),
        grid=(N // _FBLK,),
        in_specs=[
            pl.BlockSpec((NC, _FBLK, DH), lambda i: (0, i, 0)),
            pl.BlockSpec((D, D), lambda i: (0, 0)),
            pl.BlockSpec((1, D), lambda i: (0, 0)),
        ],
        out_specs=pl.BlockSpec((_FBLK, D), lambda i: (i, 0)),
        out_shape=jax.ShapeDtypeStruct((N, D), jnp.float32),
    )(p, w, b.reshape(1, D))


def kernel(x, edge_index, W1, b1, W2, b2, W3, b3):
    src = edge_index[0].astype(jnp.int32)
    dst = edge_index[1].astype(jnp.int32)
    pad = E_PAD - E
    src = jnp.concatenate([src, jnp.zeros((pad,), jnp.int32)]).reshape(E_PAD // C, C)
    dst = jnp.concatenate([dst, jnp.full((pad,), N, jnp.int32)]).reshape(E_PAD // C, C)

    xp = jnp.pad(x, ((0, ROWS - N), (0, 0)))
    xs = xp.reshape(ROWS, NC, DH).transpose(1, 0, 2).astype(jnp.bfloat16)
    xb = xs.reshape(NC * ROWS, DH)
    xb = _interleave(xb).reshape(NC, ROWS, DH)

    p = _sc_aggregate(xb, src, dst)
    h = _dense_relu(p, W1, b1)
    p = _sc_aggregate(h, src, dst)
    h = _dense_relu(p, W2, b2)
    p = _sc_aggregate(h, src, dst)
    return _dense_logsoftmax(p, W3, b3)


# speedup vs baseline: 1.1479x; 1.1479x over previous
import functools

import jax
import jax.numpy as jnp
from jax import lax
from jax.experimental import pallas as pl
from jax.experimental.pallas import tpu as pltpu
from jax.experimental.pallas import tpu_sc as plsc

N = 10000
D = 128
E = 320000

NC = 2
NS = 16
DH = D // NC
C = 128

EPW = 20480
E_PAD = EPW * NS
NCHUNK = EPW // C
NBUF = 4
ROWS = 10240
ROWS_PER_SUB = ROWS // NS
ZROWS = 128

_mesh = plsc.VectorSubcoreMesh(
    core_axis_name="c", subcore_axis_name="s", num_cores=NC, num_subcores=NS
)


@functools.partial(
    pl.kernel,
    out_type=jax.ShapeDtypeStruct((NC, ROWS, DH), jnp.float32),
    mesh=_mesh,
    scratch_types=[
        pltpu.VMEM((NCHUNK, C), jnp.int32),
        pltpu.VMEM((NCHUNK, C), jnp.int32),
        pltpu.VMEM((NBUF, C, DH), jnp.float32),
        pltpu.VMEM((ZROWS, DH), jnp.float32),
        pltpu.VMEM_SHARED((ROWS, DH), jnp.float32),
        pltpu.SemaphoreType.DMA,
        pltpu.SemaphoreType.DMA,
        pltpu.SemaphoreType.DMA,
        pltpu.SemaphoreType.DMA,
        pltpu.SemaphoreType.DMA,
        pltpu.SemaphoreType.DMA,
    ],
    compiler_params=pltpu.CompilerParams(use_tc_tiling_on_sc=False),
)
def _sc_aggregate(h_hbm, src_hbm, dst_hbm, out_hbm,
                  idxs_v, idxd_v, rows_v, zero_v, acc_sh,
                  sem0, sem1, sem2, sem3, isem_s, isem_d):
    sems = (sem0, sem1, sem2, sem3)
    ci = lax.axis_index("c")
    si = lax.axis_index("s")
    h_half = h_hbm.at[ci]
    sb0 = si * NCHUNK

    idx_s_cp = pltpu.make_async_copy(
        src_hbm.at[pl.ds(sb0, NCHUNK)], idxs_v, isem_s)
    idx_d_cp = pltpu.make_async_copy(
        dst_hbm.at[pl.ds(sb0, NCHUNK)], idxd_v, isem_d)
    idx_s_cp.start()
    idx_d_cp.start()

    zeros16 = jnp.zeros((16,), jnp.float32)

    @pl.loop(0, ZROWS)
    def _(i):
        @pl.loop(0, DH // 16)
        def _(j):
            zero_v[i, pl.ds(j * 16, 16)] = zeros16

    row0 = si * ROWS_PER_SUB

    @pl.loop(0, ROWS_PER_SUB // ZROWS)
    def _(k):
        pltpu.sync_copy(zero_v, acc_sh.at[pl.ds(row0 + k * ZROWS, ZROWS)])

    plsc.subcore_barrier()
    idx_s_cp.wait()
    idx_d_cp.wait()

    def start_gather(b, t):
        pltpu.async_copy(h_half.at[idxs_v.at[t]], rows_v.at[b], sems[b])

    def finish_chunk(b, t):
        pltpu.make_async_copy(
            h_half.at[idxs_v.at[t]], rows_v.at[b], sems[b]).wait()
        pltpu.sync_copy(rows_v.at[b], acc_sh.at[idxd_v.at[t]], add=True)

    for b in range(NBUF - 1):
        start_gather(b, b)

    @pl.loop(0, NCHUNK // NBUF)
    def _(g):
        t = g * NBUF
        for k in range(NBUF):
            f = t + k + NBUF - 1

            @pl.when(f < NCHUNK)
            def _():
                start_gather((k + NBUF - 1) % NBUF, f)

            finish_chunk(k, t + k)

    plsc.subcore_barrier()
    pltpu.sync_copy(acc_sh.at[pl.ds(row0, ROWS_PER_SUB)],
                    out_hbm.at[ci].at[pl.ds(row0, ROWS_PER_SUB)])


_BLK = 1024


def _dense_body(act, p_ref, w_ref, b_ref, o_ref):
    y = jnp.concatenate([p_ref[0], p_ref[1]], axis=1)
    y = lax.dot_general(y, w_ref[...], (((1,), (0,)), ((), ())),
                        precision=lax.Precision.HIGHEST,
                        preferred_element_type=jnp.float32)
    y = y + b_ref[...]
    if act == "relu":
        y = jnp.maximum(y, 0.0)
        o_ref[0], o_ref[1] = y[:, :DH], y[:, DH:]
    else:
        m = jnp.max(y, axis=-1, keepdims=True)
        s = jnp.sum(jnp.exp(y - m), axis=-1, keepdims=True)
        o_ref[...] = y - (m + jnp.log(s))


def _dense_relu(p, w, b):
    return pl.pallas_call(
        functools.partial(_dense_body, "relu"),
        grid=(ROWS // _BLK,),
        in_specs=[
            pl.BlockSpec((NC, _BLK, DH), lambda i: (0, i, 0)),
            pl.BlockSpec((D, D), lambda i: (0, 0)),
            pl.BlockSpec((1, D), lambda i: (0, 0)),
        ],
        out_specs=pl.BlockSpec((NC, _BLK, DH), lambda i: (0, i, 0)),
        out_shape=jax.ShapeDtypeStruct((NC, ROWS, DH), jnp.float32),
    )(p, w, b.reshape(1, D))


_FBLK = 1000


def _dense_logsoftmax(p, w, b):
    return pl.pallas_call(
        functools.partial(_dense_body, "logsoftmax"),
        grid=(N // _FBLK,),
        in_specs=[
            pl.BlockSpec((NC, _FBLK, DH), lambda i: (0, i, 0)),
            pl.BlockSpec((D, D), lambda i: (0, 0)),
            pl.BlockSpec((1, D), lambda i: (0, 0)),
        ],
        out_specs=pl.BlockSpec((_FBLK, D), lambda i: (i, 0)),
        out_shape=jax.ShapeDtypeStruct((N, D), jnp.float32),
    )(p, w, b.reshape(1, D))


def kernel(x, edge_index, W1, b1, W2, b2, W3, b3):
    src = edge_index[0].astype(jnp.int32)
    dst = edge_index[1].astype(jnp.int32)
    pad = E_PAD - E
    src = jnp.concatenate([src, jnp.zeros((pad,), jnp.int32)]).reshape(E_PAD // C, C)
    dst = jnp.concatenate([dst, jnp.full((pad,), N, jnp.int32)]).reshape(E_PAD // C, C)

    xp = jnp.pad(x, ((0, ROWS - N), (0, 0)))
    xs = xp.reshape(ROWS, NC, DH).transpose(1, 0, 2)

    p = _sc_aggregate(xs, src, dst)
    h = _dense_relu(p, W1, b1)
    p = _sc_aggregate(h, src, dst)
    h = _dense_relu(p, W2, b2)
    p = _sc_aggregate(h, src, dst)
    return _dense_logsoftmax(p, W3, b3)
